# baseline (speedup 1.0000x reference)
import jax
import jax.numpy as jnp
from jax import lax
from jax.experimental import pallas as pl
from jax.experimental.pallas import tpu as pltpu
from jax.experimental.pallas import tpu_sc as plsc

B = 128
V = 100000
K = 50

NC = 2
NS = 16
NW = NC * NS
RPW = B // NW

CHUNK = 20000
NCHUNK = V // CHUNK
GROUP = 10
GPC = CHUNK // (16 * GROUP)

CAP = CHUNK + 192

BIG_I = 1 << 30
NEG = -jnp.inf


def _sc_body(flat_hbm, gpad_hbm, out_hbm, buf0, buf1, bufp, valbuf, idxbuf,
             accv, outval, outidx, gvec, tokrow, sem0, sem1, semp):
    iota16 = lax.iota(jnp.int32, 16)
    NEGV = jnp.full((16,), NEG, jnp.float32)
    ZI = jnp.full((16,), 0, jnp.int32)
    bufs = (buf0, buf1)
    sems = (sem0, sem1)

    def bmax(v):
        for s in (8, 4, 2, 1):
            v = jnp.maximum(v, v[iota16 ^ s])
        return v

    def bmin(v):
        for s in (8, 4, 2, 1):
            v = jnp.minimum(v, v[iota16 ^ s])
        return v

    def bor(v):
        for s in (8, 4, 2, 1):
            v = jnp.bitwise_or(v, v[iota16 ^ s])
        return v

    def splat_f(x):
        return jnp.full((16,), x, jnp.float32)

    def splat_i(x):
        return jnp.full((16,), x, jnp.int32)

    wid = lax.axis_index("s") * NC + lax.axis_index("c")

    tokrow[...] = ZI
    pltpu.async_copy(
        flat_hbm.at[pl.ds(wid * RPW * V, CHUNK)], bufp, semp)

    def select50(wp):
        for q in range(4):
            outval[pl.ds(q * 16, 16)] = NEGV
        nb = (wp // 16 + 3) // 4

        def round_body(i, _):
            def sweep(jb, carry):
                m, pos = carry
                base = jb * 64
                for u in range(4):
                    v = valbuf[pl.ds(base + u * 16, 16)]
                    upd = v > m
                    m = jnp.maximum(m, v)
                    pos = jnp.where(upd, splat_i(base + u * 16) + iota16,
                                    pos)
                return (m, pos)
            m, pos = lax.fori_loop(0, nb, sweep, (NEGV, splat_i(BIG_I)))
            gmv = bmax(m)
            p = bmin(jnp.where(m == gmv, pos, splat_i(BIG_I)))[0]

            base = (p // 16) * 16
            off = p - base
            vw = valbuf[pl.ds(base, 16)]
            iw = idxbuf[pl.ds(base, 16)]
            idxs = bmax(jnp.where(iota16 == off, iw, ZI))
            valbuf[pl.ds(base, 16)] = jnp.where(iota16 == off, NEGV, vw)

            obase = (i // 16) * 16
            ooff = i - obase
            ow = outval[pl.ds(obase, 16)]
            outval[pl.ds(obase, 16)] = jnp.where(iota16 == ooff, gmv, ow)
            oi = outidx[pl.ds(obase, 16)]
            outidx[pl.ds(obase, 16)] = jnp.where(iota16 == ooff, idxs, oi)
            return 0

        lax.fori_loop(0, K, round_body, 0)

        for q in range(4):
            valbuf[pl.ds(q * 16, 16)] = outval[pl.ds(q * 16, 16)]
            idxbuf[pl.ds(q * 16, 16)] = outidx[pl.ds(q * 16, 16)]
        for q in range(4, 7):
            valbuf[pl.ds(q * 16, 16)] = NEGV
        t50 = outval[pl.ds(48, 16)][1]
        return t50, jnp.int32(64)

    def scan_chunk(bufref, col_base, t, wp):
        tv = splat_f(t)

        def gbody(g, wp):
            goff = g * (GROUP * 16)
            vs = [bufref[pl.ds(goff + 16 * j, 16)] for j in range(GROUP)]
            m = vs[0]
            for v in vs[1:]:
                m = jnp.maximum(m, v)
            gmax = bmax(m)[0]

            def do_append(wp):
                bm = ZI
                for j in range(GROUP):
                    bm = jnp.bitwise_or(
                        bm, jnp.where(vs[j] >= tv, splat_i(1 << j), ZI))
                bms = bor(bm)[0]
                for j in range(GROUP):
                    def store_j(w, j=j):
                        base = jnp.minimum(w, CAP - 16)
                        valbuf[pl.ds(base, 16)] = jnp.where(
                            vs[j] >= tv, vs[j], NEGV)
                        idxbuf[pl.ds(base, 16)] = (
                            splat_i(col_base + goff + j * 16) + iota16)
                        return w + 16
                    wp = lax.cond(
                        jnp.bitwise_and(
                            lax.shift_right_logical(bms, j), 1) != 0,
                        store_j, lambda w: w, wp)
                return wp

            return lax.cond(gmax >= t, do_append, lambda w: w, wp)
        return lax.fori_loop(0, GPC, gbody, wp)

    def row_body(i, _):
        row = wid * RPW + i
        rbase = row * V

        pltpu.sync_copy(gpad_hbm.at[pl.ds(row * 64, 64)], gvec)
        pltpu.make_async_copy(
            flat_hbm.at[pl.ds(rbase, CHUNK)], bufp, semp).wait()
        h1 = pltpu.async_copy(
            flat_hbm.at[pl.ds(rbase + CHUNK, CHUNK)], buf1, sem1)

        for a in range(16):
            def wmax(jj, m, a=a):
                base = a * 1216 + jj * 64
                for u in range(4):
                    m = jnp.maximum(m, bufp[pl.ds(base + u * 16, 16)])
                return m
            accv[pl.ds(a * 16, 16)] = lax.fori_loop(0, 19, wmax, NEGV)

        def maxround(ii, _):
            def asweep(jb, carry):
                m, pos = carry
                base = jb * 64
                for u in range(4):
                    v = accv[pl.ds(base + u * 16, 16)]
                    upd = v > m
                    m = jnp.maximum(m, v)
                    pos = jnp.where(upd, splat_i(base + u * 16) + iota16,
                                    pos)
                return (m, pos)
            m, pos = lax.fori_loop(0, 4, asweep, (NEGV, splat_i(BIG_I)))
            gmv = bmax(m)
            p = bmin(jnp.where(m == gmv, pos, splat_i(BIG_I)))[0]
            pb = (p // 16) * 16
            w = accv[pl.ds(pb, 16)]
            accv[pl.ds(pb, 16)] = jnp.where(
                iota16 == p - pb, NEGV, w)
            return gmv[0]
        t = lax.fori_loop(0, K, maxround, NEG)

        wp = jnp.int32(0)
        handles = [None] * NCHUNK
        handles[1] = h1
        for c in range(NCHUNK):
            if c > 0:
                handles[c].wait()
            if c + 1 < NCHUNK:
                handles[c + 1] = pltpu.async_copy(
                    flat_hbm.at[pl.ds(rbase + (c + 1) * CHUNK, CHUNK)],
                    bufs[(c + 1) % 2], sems[(c + 1) % 2])
            wp = scan_chunk(bufp if c == 0 else bufs[c % 2],
                            c * CHUNK, t, wp)
            if c == 0:
                rnext = jnp.minimum(row + 1, B - 1) * V
                pltpu.async_copy(
                    flat_hbm.at[pl.ds(rnext, CHUNK)], bufp, semp)
            for q in range(3):
                valbuf[pl.ds(jnp.minimum(wp + q * 16, CAP - 16),
                             16)] = NEGV
            t50, wp = select50(wp)
            t = jnp.maximum(t, t50)

        def smax(q, carry):
            m, pos = carry
            s = outval[pl.ds(q * 16, 16)] + gvec[pl.ds(q * 16, 16)]
            upd = s > m
            m = jnp.maximum(m, s)
            pos = jnp.where(upd, splat_i(0) + q * 16 + iota16, pos)
            return (m, pos)
        m, pos = lax.fori_loop(0, 4, smax, (NEGV, splat_i(BIG_I)))
        gmv = bmax(m)
        p = bmin(jnp.where(m == gmv, pos, splat_i(BIG_I)))[0]
        base = (p // 16) * 16
        iw = outidx[pl.ds(base, 16)]
        tokv = bmax(jnp.where(iota16 == p - base, iw, ZI))
        tokrow[...] = jnp.where(iota16 == splat_i(i), tokv, tokrow[...])
        return 0

    lax.fori_loop(0, RPW, row_body, 0)
    pltpu.sync_copy(tokrow, out_hbm.at[pl.ds(wid * 16, 16)])


def kernel(logits, input_ids):
    flat = logits.reshape(-1)
    g = jax.random.gumbel(jax.random.key(42), (B, K), jnp.float32)
    gpad = jnp.concatenate(
        [g, jnp.full((B, 14), -jnp.inf, jnp.float32)], axis=1).reshape(-1)

    mesh = plsc.VectorSubcoreMesh(
        core_axis_name="c", subcore_axis_name="s",
        num_cores=NC, num_subcores=NS)
    toks = pl.kernel(
        _sc_body,
        out_type=jax.ShapeDtypeStruct((NW * 16,), jnp.int32),
        mesh=mesh,
        scratch_types=[
            pltpu.VMEM((CHUNK,), jnp.float32),
            pltpu.VMEM((CHUNK,), jnp.float32),
            pltpu.VMEM((CHUNK,), jnp.float32),
            pltpu.VMEM((CAP,), jnp.float32),
            pltpu.VMEM((CAP,), jnp.int32),
            pltpu.VMEM((256,), jnp.float32),
            pltpu.VMEM((64,), jnp.float32),
            pltpu.VMEM((64,), jnp.int32),
            pltpu.VMEM((64,), jnp.float32),
            pltpu.VMEM((16,), jnp.int32),
            pltpu.SemaphoreType.DMA,
            pltpu.SemaphoreType.DMA,
            pltpu.SemaphoreType.DMA,
        ],
    )(flat, gpad)

    tokens = toks.reshape(NW, 16)[:, :RPW].reshape(B)
    return jnp.concatenate(
        [input_ids, tokens[:, None].astype(input_ids.dtype)], axis=1)


# speedup vs baseline: 1.0338x; 1.0338x over previous
import jax
import jax.numpy as jnp
from jax import lax
from jax.experimental import pallas as pl
from jax.experimental.pallas import tpu as pltpu
from jax.experimental.pallas import tpu_sc as plsc

B = 128
V = 100000
K = 50

NC = 2
NS = 16
NW = NC * NS
RPW = B // NW

CHUNK = 20000
NCHUNK = V // CHUNK
GROUP = 10
GPC = CHUNK // (16 * GROUP)

CAP = 2 * CHUNK + 192

BIG_I = 1 << 30
NEG = -jnp.inf


def _sc_body(flat_hbm, gpad_hbm, out_hbm, buf0, buf1, valbuf, idxbuf,
             accv, outval, outidx, gvec, tokrow, sem0, sem1):
    iota16 = lax.iota(jnp.int32, 16)
    NEGV = jnp.full((16,), NEG, jnp.float32)
    ZI = jnp.full((16,), 0, jnp.int32)
    bufs = (buf0, buf1)
    sems = (sem0, sem1)

    def bmax(v):
        for s in (8, 4, 2, 1):
            v = jnp.maximum(v, v[iota16 ^ s])
        return v

    def bmin(v):
        for s in (8, 4, 2, 1):
            v = jnp.minimum(v, v[iota16 ^ s])
        return v

    def bor(v):
        for s in (8, 4, 2, 1):
            v = jnp.bitwise_or(v, v[iota16 ^ s])
        return v

    def splat_f(x):
        return jnp.full((16,), x, jnp.float32)

    def splat_i(x):
        return jnp.full((16,), x, jnp.int32)

    wid = lax.axis_index("s") * NC + lax.axis_index("c")

    tokrow[...] = ZI

    def select50(wp):
        for q in range(4):
            outval[pl.ds(q * 16, 16)] = NEGV
        nb = (wp // 16 + 3) // 4

        def round_body(i, _):
            def sweep(jb, carry):
                m, pos = carry
                base = jb * 64
                for u in range(4):
                    v = valbuf[pl.ds(base + u * 16, 16)]
                    upd = v > m
                    m = jnp.maximum(m, v)
                    pos = jnp.where(upd, splat_i(base + u * 16) + iota16,
                                    pos)
                return (m, pos)
            m, pos = lax.fori_loop(0, nb, sweep, (NEGV, splat_i(BIG_I)))
            gmv = bmax(m)
            p = bmin(jnp.where(m == gmv, pos, splat_i(BIG_I)))[0]

            base = (p // 16) * 16
            off = p - base
            vw = valbuf[pl.ds(base, 16)]
            iw = idxbuf[pl.ds(base, 16)]
            idxs = bmax(jnp.where(iota16 == off, iw, ZI))
            valbuf[pl.ds(base, 16)] = jnp.where(iota16 == off, NEGV, vw)

            obase = (i // 16) * 16
            ooff = i - obase
            ow = outval[pl.ds(obase, 16)]
            outval[pl.ds(obase, 16)] = jnp.where(iota16 == ooff, gmv, ow)
            oi = outidx[pl.ds(obase, 16)]
            outidx[pl.ds(obase, 16)] = jnp.where(iota16 == ooff, idxs, oi)
            return 0

        lax.fori_loop(0, K, round_body, 0)

        for q in range(4):
            valbuf[pl.ds(q * 16, 16)] = outval[pl.ds(q * 16, 16)]
            idxbuf[pl.ds(q * 16, 16)] = outidx[pl.ds(q * 16, 16)]
        for q in range(4, 7):
            valbuf[pl.ds(q * 16, 16)] = NEGV
        t50 = outval[pl.ds(48, 16)][1]
        return t50, jnp.int32(64)

    def scan_chunk(bufref, col_base, t, wp):
        tv = splat_f(t)

        def gbody(g, wp):
            goff = g * (GROUP * 16)
            vs = [bufref[pl.ds(goff + 16 * j, 16)] for j in range(GROUP)]
            m = vs[0]
            for v in vs[1:]:
                m = jnp.maximum(m, v)
            gmax = bmax(m)[0]

            def do_append(wp):
                bm = ZI
                for j in range(GROUP):
                    bm = jnp.bitwise_or(
                        bm, jnp.where(vs[j] >= tv, splat_i(1 << j), ZI))
                bms = bor(bm)[0]
                for j in range(GROUP):
                    def store_j(w, j=j):
                        base = jnp.minimum(w, CAP - 16)
                        valbuf[pl.ds(base, 16)] = jnp.where(
                            vs[j] >= tv, vs[j], NEGV)
                        idxbuf[pl.ds(base, 16)] = (
                            splat_i(col_base + goff + j * 16) + iota16)
                        return w + 16
                    wp = lax.cond(
                        jnp.bitwise_and(
                            lax.shift_right_logical(bms, j), 1) != 0,
                        store_j, lambda w: w, wp)
                return wp

            return lax.cond(gmax >= t, do_append, lambda w: w, wp)
        return lax.fori_loop(0, GPC, gbody, wp)

    def row_body(i, _):
        row = wid * RPW + i
        rbase = row * V

        h0 = pltpu.async_copy(flat_hbm.at[pl.ds(rbase, CHUNK)], buf0, sem0)
        pltpu.sync_copy(gpad_hbm.at[pl.ds(row * 64, 64)], gvec)
        h0.wait()
        h1 = pltpu.async_copy(
            flat_hbm.at[pl.ds(rbase + CHUNK, CHUNK)], buf1, sem1)

        for a in range(16):
            def wmax(jj, m, a=a):
                base = a * 1216 + jj * 64
                for u in range(4):
                    m = jnp.maximum(m, buf0[pl.ds(base + u * 16, 16)])
                return m
            accv[pl.ds(a * 16, 16)] = lax.fori_loop(0, 19, wmax, NEGV)

        def maxround(ii, _):
            def asweep(jb, carry):
                m, pos = carry
                base = jb * 64
                for u in range(4):
                    v = accv[pl.ds(base + u * 16, 16)]
                    upd = v > m
                    m = jnp.maximum(m, v)
                    pos = jnp.where(upd, splat_i(base + u * 16) + iota16,
                                    pos)
                return (m, pos)
            m, pos = lax.fori_loop(0, 4, asweep, (NEGV, splat_i(BIG_I)))
            gmv = bmax(m)
            p = bmin(jnp.where(m == gmv, pos, splat_i(BIG_I)))[0]
            pb = (p // 16) * 16
            w = accv[pl.ds(pb, 16)]
            accv[pl.ds(pb, 16)] = jnp.where(
                iota16 == p - pb, NEGV, w)
            return gmv[0]
        t = lax.fori_loop(0, K, maxround, NEG)

        wp = jnp.int32(0)
        handles = [None] * NCHUNK
        handles[1] = h1
        for c in range(NCHUNK):
            if c > 0:
                handles[c].wait()
            if c + 1 < NCHUNK:
                handles[c + 1] = pltpu.async_copy(
                    flat_hbm.at[pl.ds(rbase + (c + 1) * CHUNK, CHUNK)],
                    bufs[(c + 1) % 2], sems[(c + 1) % 2])
            wp = scan_chunk(bufs[c % 2], c * CHUNK, t, wp)
            if c in (0, 2, 4):
                for q in range(3):
                    valbuf[pl.ds(jnp.minimum(wp + q * 16, CAP - 16),
                                 16)] = NEGV
                t50, wp = select50(wp)
                t = jnp.maximum(t, t50)

        def smax(q, carry):
            m, pos = carry
            s = outval[pl.ds(q * 16, 16)] + gvec[pl.ds(q * 16, 16)]
            upd = s > m
            m = jnp.maximum(m, s)
            pos = jnp.where(upd, splat_i(0) + q * 16 + iota16, pos)
            return (m, pos)
        m, pos = lax.fori_loop(0, 4, smax, (NEGV, splat_i(BIG_I)))
        gmv = bmax(m)
        p = bmin(jnp.where(m == gmv, pos, splat_i(BIG_I)))[0]
        base = (p // 16) * 16
        iw = outidx[pl.ds(base, 16)]
        tokv = bmax(jnp.where(iota16 == p - base, iw, ZI))
        tokrow[...] = jnp.where(iota16 == splat_i(i), tokv, tokrow[...])
        return 0

    lax.fori_loop(0, RPW, row_body, 0)
    pltpu.sync_copy(tokrow, out_hbm.at[pl.ds(wid * 16, 16)])


def kernel(logits, input_ids):
    flat = logits.reshape(-1)
    g = jax.random.gumbel(jax.random.key(42), (B, K), jnp.float32)
    gpad = jnp.concatenate(
        [g, jnp.full((B, 14), -jnp.inf, jnp.float32)], axis=1).reshape(-1)

    mesh = plsc.VectorSubcoreMesh(
        core_axis_name="c", subcore_axis_name="s",
        num_cores=NC, num_subcores=NS)
    toks = pl.kernel(
        _sc_body,
        out_type=jax.ShapeDtypeStruct((NW * 16,), jnp.int32),
        mesh=mesh,
        scratch_types=[
            pltpu.VMEM((CHUNK,), jnp.float32),
            pltpu.VMEM((CHUNK,), jnp.float32),
            pltpu.VMEM((CAP,), jnp.float32),
            pltpu.VMEM((CAP,), jnp.int32),
            pltpu.VMEM((256,), jnp.float32),
            pltpu.VMEM((64,), jnp.float32),
            pltpu.VMEM((64,), jnp.int32),
            pltpu.VMEM((64,), jnp.float32),
            pltpu.VMEM((16,), jnp.int32),
            pltpu.SemaphoreType.DMA,
            pltpu.SemaphoreType.DMA,
        ],
    )(flat, gpad)

    tokens = toks.reshape(NW, 16)[:, :RPW].reshape(B)
    return jnp.concatenate(
        [input_ids, tokens[:, None].astype(input_ids.dtype)], axis=1)


# speedup vs baseline: 1.1094x; 1.0731x over previous
import jax
import jax.numpy as jnp
from jax import lax
from jax.experimental import pallas as pl
from jax.experimental.pallas import tpu as pltpu
from jax.experimental.pallas import tpu_sc as plsc

B = 128
V = 100000
K = 50

NC = 2
NS = 16
NW = NC * NS
RPW = B // NW

CHUNK = 20000
NCHUNK = V // CHUNK
GROUP = 25
GPC = CHUNK // (16 * GROUP)

CAP = 2 * CHUNK + 192

BIG_I = 1 << 30
NEG = -jnp.inf


def _sc_body(flat_hbm, gpad_hbm, out_hbm, buf0, buf1, valbuf, idxbuf,
             accv, outval, outidx, gvec, tokrow, sem0, sem1):
    iota16 = lax.iota(jnp.int32, 16)
    NEGV = jnp.full((16,), NEG, jnp.float32)
    ZI = jnp.full((16,), 0, jnp.int32)
    bufs = (buf0, buf1)
    sems = (sem0, sem1)

    def bmax(v):
        for s in (8, 4, 2, 1):
            v = jnp.maximum(v, v[iota16 ^ s])
        return v

    def bmin(v):
        for s in (8, 4, 2, 1):
            v = jnp.minimum(v, v[iota16 ^ s])
        return v

    def bor(v):
        for s in (8, 4, 2, 1):
            v = jnp.bitwise_or(v, v[iota16 ^ s])
        return v

    def splat_f(x):
        return jnp.full((16,), x, jnp.float32)

    def splat_i(x):
        return jnp.full((16,), x, jnp.int32)

    wid = lax.axis_index("s") * NC + lax.axis_index("c")

    tokrow[...] = ZI

    def select50(wp):
        for q in range(4):
            outval[pl.ds(q * 16, 16)] = NEGV
        nb = (wp // 16 + 3) // 4

        def round_body(i, _):
            def sweep(jb, carry):
                m, pos = carry
                base = jb * 64
                for u in range(4):
                    v = valbuf[pl.ds(base + u * 16, 16)]
                    upd = v > m
                    m = jnp.maximum(m, v)
                    pos = jnp.where(upd, splat_i(base + u * 16) + iota16,
                                    pos)
                return (m, pos)
            m, pos = lax.fori_loop(0, nb, sweep, (NEGV, splat_i(BIG_I)))
            gmv = bmax(m)
            p = bmin(jnp.where(m == gmv, pos, splat_i(BIG_I)))[0]

            base = (p // 16) * 16
            off = p - base
            vw = valbuf[pl.ds(base, 16)]
            iw = idxbuf[pl.ds(base, 16)]
            idxs = bmax(jnp.where(iota16 == off, iw, ZI))
            valbuf[pl.ds(base, 16)] = jnp.where(iota16 == off, NEGV, vw)

            obase = (i // 16) * 16
            ooff = i - obase
            ow = outval[pl.ds(obase, 16)]
            outval[pl.ds(obase, 16)] = jnp.where(iota16 == ooff, gmv, ow)
            oi = outidx[pl.ds(obase, 16)]
            outidx[pl.ds(obase, 16)] = jnp.where(iota16 == ooff, idxs, oi)
            return 0

        lax.fori_loop(0, K, round_body, 0)

        for q in range(4):
            valbuf[pl.ds(q * 16, 16)] = outval[pl.ds(q * 16, 16)]
            idxbuf[pl.ds(q * 16, 16)] = outidx[pl.ds(q * 16, 16)]
        for q in range(4, 7):
            valbuf[pl.ds(q * 16, 16)] = NEGV
        t50 = outval[pl.ds(48, 16)][1]
        return t50, jnp.int32(64)

    def scan_chunk(bufref, col_base, t, wp):
        tv = splat_f(t)

        def gbody(g, wp):
            goff = g * (GROUP * 16)
            vs = [bufref[pl.ds(goff + 16 * j, 16)] for j in range(GROUP)]
            m = vs[0]
            for v in vs[1:]:
                m = jnp.maximum(m, v)
            gmax = bmax(m)[0]

            def do_append(wp):
                bm = ZI
                for j in range(GROUP):
                    bm = jnp.bitwise_or(
                        bm, jnp.where(vs[j] >= tv, splat_i(1 << j), ZI))
                bms = bor(bm)[0]
                for j in range(GROUP):
                    def store_j(w, j=j):
                        base = jnp.minimum(w, CAP - 16)
                        valbuf[pl.ds(base, 16)] = jnp.where(
                            vs[j] >= tv, vs[j], NEGV)
                        idxbuf[pl.ds(base, 16)] = (
                            splat_i(col_base + goff + j * 16) + iota16)
                        return w + 16
                    wp = lax.cond(
                        jnp.bitwise_and(
                            lax.shift_right_logical(bms, j), 1) != 0,
                        store_j, lambda w: w, wp)
                return wp

            return lax.cond(gmax >= t, do_append, lambda w: w, wp)
        return lax.fori_loop(0, GPC, gbody, wp)

    def row_body(i, _):
        row = wid * RPW + i
        rbase = row * V

        h0 = pltpu.async_copy(flat_hbm.at[pl.ds(rbase, CHUNK)], buf0, sem0)
        pltpu.sync_copy(gpad_hbm.at[pl.ds(row * 64, 64)], gvec)
        h0.wait()
        h1 = pltpu.async_copy(
            flat_hbm.at[pl.ds(rbase + CHUNK, CHUNK)], buf1, sem1)

        for a in range(16):
            def wmax(jj, m, a=a):
                base = a * 1216 + jj * 64
                for u in range(4):
                    m = jnp.maximum(m, buf0[pl.ds(base + u * 16, 16)])
                return m
            accv[pl.ds(a * 16, 16)] = lax.fori_loop(0, 19, wmax, NEGV)

        def maxround(ii, _):
            def asweep(jb, carry):
                m, pos = carry
                base = jb * 64
                for u in range(4):
                    v = accv[pl.ds(base + u * 16, 16)]
                    upd = v > m
                    m = jnp.maximum(m, v)
                    pos = jnp.where(upd, splat_i(base + u * 16) + iota16,
                                    pos)
                return (m, pos)
            m, pos = lax.fori_loop(0, 4, asweep, (NEGV, splat_i(BIG_I)))
            gmv = bmax(m)
            p = bmin(jnp.where(m == gmv, pos, splat_i(BIG_I)))[0]
            pb = (p // 16) * 16
            w = accv[pl.ds(pb, 16)]
            accv[pl.ds(pb, 16)] = jnp.where(
                iota16 == p - pb, NEGV, w)
            return gmv[0]
        t = lax.fori_loop(0, K, maxround, NEG)

        wp = jnp.int32(0)
        handles = [None] * NCHUNK
        handles[1] = h1
        for c in range(NCHUNK):
            if c > 0:
                handles[c].wait()
            if c + 1 < NCHUNK:
                handles[c + 1] = pltpu.async_copy(
                    flat_hbm.at[pl.ds(rbase + (c + 1) * CHUNK, CHUNK)],
                    bufs[(c + 1) % 2], sems[(c + 1) % 2])
            wp = scan_chunk(bufs[c % 2], c * CHUNK, t, wp)
            if c in (0, 2, 4):
                for q in range(3):
                    valbuf[pl.ds(jnp.minimum(wp + q * 16, CAP - 16),
                                 16)] = NEGV
                t50, wp = select50(wp)
                t = jnp.maximum(t, t50)

        def smax(q, carry):
            m, pos = carry
            s = outval[pl.ds(q * 16, 16)] + gvec[pl.ds(q * 16, 16)]
            upd = s > m
            m = jnp.maximum(m, s)
            pos = jnp.where(upd, splat_i(0) + q * 16 + iota16, pos)
            return (m, pos)
        m, pos = lax.fori_loop(0, 4, smax, (NEGV, splat_i(BIG_I)))
        gmv = bmax(m)
        p = bmin(jnp.where(m == gmv, pos, splat_i(BIG_I)))[0]
        base = (p // 16) * 16
        iw = outidx[pl.ds(base, 16)]
        tokv = bmax(jnp.where(iota16 == p - base, iw, ZI))
        tokrow[...] = jnp.where(iota16 == splat_i(i), tokv, tokrow[...])
        return 0

    lax.fori_loop(0, RPW, row_body, 0)
    pltpu.sync_copy(tokrow, out_hbm.at[pl.ds(wid * 16, 16)])


def kernel(logits, input_ids):
    flat = logits.reshape(-1)
    g = jax.random.gumbel(jax.random.key(42), (B, K), jnp.float32)
    gpad = jnp.concatenate(
        [g, jnp.full((B, 14), -jnp.inf, jnp.float32)], axis=1).reshape(-1)

    mesh = plsc.VectorSubcoreMesh(
        core_axis_name="c", subcore_axis_name="s",
        num_cores=NC, num_subcores=NS)
    toks = pl.kernel(
        _sc_body,
        out_type=jax.ShapeDtypeStruct((NW * 16,), jnp.int32),
        mesh=mesh,
        scratch_types=[
            pltpu.VMEM((CHUNK,), jnp.float32),
            pltpu.VMEM((CHUNK,), jnp.float32),
            pltpu.VMEM((CAP,), jnp.float32),
            pltpu.VMEM((CAP,), jnp.int32),
            pltpu.VMEM((256,), jnp.float32),
            pltpu.VMEM((64,), jnp.float32),
            pltpu.VMEM((64,), jnp.int32),
            pltpu.VMEM((64,), jnp.float32),
            pltpu.VMEM((16,), jnp.int32),
            pltpu.SemaphoreType.DMA,
            pltpu.SemaphoreType.DMA,
        ],
    )(flat, gpad)

    tokens = toks.reshape(NW, 16)[:, :RPW].reshape(B)
    return jnp.concatenate(
        [input_ids, tokens[:, None].astype(input_ids.dtype)], axis=1)
